# baseline (speedup 1.0000x reference)
import jax
import jax.numpy as jnp
from jax.experimental import pallas as pl
from jax.experimental.pallas import tpu as pltpu

_CD = jnp.bfloat16
_NT = (((1,), (1,)), ((), ()))


def _make_body(R, B, Din, D0, D1):
    r0_lo = D0 + D1
    r1_lo = r0_lo + B
    bias_lo = r1_lo + B

    def body(x_ref, *refs):
        adj_refs = refs[:R]
        pk_ref = refs[R]
        out_ref = refs[R + 1]
        b = pl.program_id(0)

        adj_c = [a_ref[0, 0].astype(_CD) for a_ref in adj_refs]

        h = x_ref[0]
        for w_lo, din, rp_lo, b_lo, D in ((0, Din, r0_lo, 0, D0),
                                          (D0, D0, r1_lo, D0, D1)):
            h_c = h.astype(_CD)
            acc = pk_ref[bias_lo:bias_lo + 1, b_lo:b_lo + D]
            relp_row = pk_ref[pl.ds(rp_lo + b, 1), :]
            for r in range(R):
                wx_r = pk_ref[w_lo:w_lo + D,
                              r * din:(r + 1) * din].astype(_CD)
                y = jax.lax.dot_general(h_c, wx_r, _NT,
                                        preferred_element_type=jnp.float32)
                y = (y + relp_row[:, r * D:(r + 1) * D]).astype(_CD)
                acc = acc + jnp.dot(adj_c[r], y,
                                    preferred_element_type=jnp.float32)
            h = jnp.maximum(acc, 0.0)
        out_ref[0] = h
    return body


def kernel(node_features, relation_features, adj, w0, b0, w1, b1):
    B, N, Din = node_features.shape
    _, R, L = relation_features.shape
    D0, D1 = w0.shape[0], w1.shape[0]

    def prep(w, in_dim):
        D = w.shape[0]
        w3 = w.reshape(D, R, in_dim + L)
        wx = w3[:, :, :in_dim].reshape(D, R * in_dim)
        relp = jnp.einsum("brl,drl->brd", relation_features, w3[:, :, in_dim:])
        return wx, relp.reshape(B, R * D)

    w0x, relp0 = prep(w0, Din)
    w1x, relp1 = prep(w1, D0)
    W = max(R * Din, R * D0, R * D1)

    def pad_w(a):
        return jnp.pad(a, ((0, 0), (0, W - a.shape[1])))

    bias_row = jnp.concatenate(
        [b0, b1, jnp.zeros((W - D0 - D1,), jnp.float32)])[None, :]
    rows = D0 + D1 + 2 * B + 1
    packed = jnp.concatenate(
        [pad_w(w0x), pad_w(w1x), pad_w(relp0), pad_w(relp1), bias_row,
         jnp.zeros(((-rows) % 8, W), jnp.float32)], axis=0)

    adj_specs = [
        pl.BlockSpec((1, 1, N, N), (lambda b, rr=r: (b, rr, 0, 0)))
        for r in range(R)
    ]
    return pl.pallas_call(
        _make_body(R, B, Din, D0, D1),
        out_shape=jax.ShapeDtypeStruct((B, N, D1), node_features.dtype),
        grid=(B,),
        in_specs=[pl.BlockSpec((1, N, Din), lambda b: (b, 0, 0))] + adj_specs + [
            pl.BlockSpec(packed.shape, lambda b: (0, 0)),
        ],
        out_specs=pl.BlockSpec((1, N, D1), lambda b: (b, 0, 0)),
        compiler_params=pltpu.CompilerParams(
            dimension_semantics=("arbitrary",),
            vmem_limit_bytes=int((64 << 20) * 0.75)),
    )(node_features, *([adj] * R), packed)


# speedup vs baseline: 1.3202x; 1.3202x over previous
import jax
import jax.numpy as jnp
from jax.experimental import pallas as pl
from jax.experimental.pallas import tpu as pltpu

_CD = jnp.bfloat16


def _make_body(R, B, D0, D1):
    def body(x_ref, *refs):
        adj_refs = refs[:R]
        pk_ref = refs[R]
        out_ref = refs[R + 1]
        din = x_ref.shape[2]
        r0, r1 = din + D0, din + D0 + 8 * B
        rb = r1 + 8 * B
        b = pl.program_id(0)

        adj_c = [a_ref[0, 0].astype(_CD) for a_ref in adj_refs]

        h = x_ref[0]
        for w_lo, w_hi, rp_lo, b_lo, D in ((0, din, r0, 0, D0),
                                           (din, din + D0, r1, D0, D1)):
            y = jnp.dot(h.astype(_CD), pk_ref[w_lo:w_hi, :],
                        preferred_element_type=jnp.float32)
            relp = pk_ref[pl.ds(rp_lo + b * 8, 1), :].astype(jnp.float32)
            y = (y + relp).astype(_CD)
            acc = jnp.dot(adj_c[0], y[:, :D], preferred_element_type=jnp.float32)
            for r in range(1, R):
                acc += jnp.dot(adj_c[r], y[:, r * D:(r + 1) * D],
                               preferred_element_type=jnp.float32)
            bias = pk_ref[rb:rb + 1, b_lo:b_lo + D].astype(jnp.float32)
            h = jnp.maximum(acc + bias, 0.0)
        out_ref[0] = h
    return body


def _prep_layer(w, rel, in_dim):
    B, R, L = rel.shape
    Dout = w.shape[0]
    w_all = jnp.transpose(w.astype(_CD)).reshape(R, in_dim + L, Dout)
    wx = jnp.transpose(w_all[:, :in_dim, :], (1, 0, 2)).reshape(in_dim, R * Dout)
    w3 = w.reshape(Dout, R, in_dim + L)
    relp = jnp.einsum("brl,drl->brd", rel, w3[:, :, in_dim:])
    return wx, relp.reshape(B, R * Dout).astype(_CD), Dout


def kernel(node_features, relation_features, adj, w0, b0, w1, b1):
    B, N, Din = node_features.shape
    _, R, _ = relation_features.shape

    wx0, relp0, D0 = _prep_layer(w0, relation_features, Din)
    wx1, relp1, D1 = _prep_layer(w1, relation_features, D0)
    W = R * max(D0, D1)

    def pad_w(a):
        return jnp.pad(a, ((0, 0), (0, W - a.shape[1])))

    bias_row = jnp.concatenate(
        [b0, b1, jnp.zeros((W - D0 - D1,), jnp.float32)]).astype(_CD)[None, :]
    rows = Din + D0 + 16 * B + 1
    packed = jnp.concatenate(
        [pad_w(wx0), pad_w(wx1),
         jnp.repeat(pad_w(relp0), 8, axis=0),
         jnp.repeat(pad_w(relp1), 8, axis=0), bias_row,
         jnp.zeros(((-rows) % 8, W), _CD)], axis=0)

    adj_specs = [
        pl.BlockSpec((1, 1, N, N), (lambda b, rr=r: (b, rr, 0, 0)))
        for r in range(R)
    ]
    return pl.pallas_call(
        _make_body(R, B, D0, D1),
        out_shape=jax.ShapeDtypeStruct((B, N, D1), node_features.dtype),
        grid=(B,),
        in_specs=[pl.BlockSpec((1, N, Din), lambda b: (b, 0, 0))] + adj_specs + [
            pl.BlockSpec(packed.shape, lambda b: (0, 0)),
        ],
        out_specs=pl.BlockSpec((1, N, D1), lambda b: (b, 0, 0)),
        compiler_params=pltpu.CompilerParams(
            dimension_semantics=("arbitrary",),
            vmem_limit_bytes=int((64 << 20) * 0.75)),
    )(node_features, *([adj] * R), packed)


# speedup vs baseline: 1.3290x; 1.0067x over previous
import jax
import jax.numpy as jnp
from jax.experimental import pallas as pl
from jax.experimental.pallas import tpu as pltpu

_CD = jnp.bfloat16


def _make_body(R, B, D0, D1):
    def body(x_ref, *refs):
        adj_refs = refs[:R]
        pk_ref = refs[R]
        out_ref = refs[R + 1]
        din = x_ref.shape[2]
        r0, r1 = din + D0, din + D0 + B
        rb = r1 + B
        b = pl.program_id(0)

        adj_c = [a_ref[0, 0].astype(_CD) for a_ref in adj_refs]

        h = x_ref[0]
        for w_lo, w_hi, rp_lo, b_lo, D in ((0, din, r0, 0, D0),
                                           (din, din + D0, r1, D0, D1)):
            wx = pk_ref[w_lo:w_hi, :].astype(_CD)
            y = jnp.dot(h.astype(_CD), wx, preferred_element_type=jnp.float32)
            y = (y + pk_ref[pl.ds(rp_lo + b, 1), :]).astype(_CD)
            acc = jnp.dot(adj_c[0], y[:, :D], preferred_element_type=jnp.float32)
            for r in range(1, R):
                acc += jnp.dot(adj_c[r], y[:, r * D:(r + 1) * D],
                               preferred_element_type=jnp.float32)
            bias = pk_ref[rb:rb + 1, b_lo:b_lo + D]
            h = jnp.maximum(acc + bias, 0.0)
        out_ref[0] = h
    return body


def _prep_layer(w, rel, in_dim):
    B, R, L = rel.shape
    Dout = w.shape[0]
    w_all = jnp.transpose(w).reshape(R, in_dim + L, Dout)
    wx = jnp.transpose(w_all[:, :in_dim, :], (1, 0, 2)).reshape(in_dim, R * Dout)
    relp = jnp.einsum("brl,rld->brd", rel, w_all[:, in_dim:, :])
    return wx, relp.reshape(B, R * Dout), Dout


def kernel(node_features, relation_features, adj, w0, b0, w1, b1):
    B, N, Din = node_features.shape
    _, R, _ = relation_features.shape

    wx0, relp0, D0 = _prep_layer(w0, relation_features, Din)
    wx1, relp1, D1 = _prep_layer(w1, relation_features, D0)
    W = R * max(D0, D1)

    def pad_w(a):
        return jnp.pad(a, ((0, 0), (0, W - a.shape[1])))

    bias_row = jnp.concatenate(
        [b0, b1, jnp.zeros((W - D0 - D1,), jnp.float32)])[None, :]
    rows = Din + D0 + 2 * B + 1
    packed = jnp.concatenate(
        [pad_w(wx0), pad_w(wx1), pad_w(relp0), pad_w(relp1), bias_row,
         jnp.zeros(((-rows) % 8, W), jnp.float32)], axis=0)

    adj_specs = [
        pl.BlockSpec((1, 1, N, N), (lambda b, rr=r: (b, rr, 0, 0)))
        for r in range(R)
    ]
    return pl.pallas_call(
        _make_body(R, B, D0, D1),
        out_shape=jax.ShapeDtypeStruct((B, N, D1), node_features.dtype),
        grid=(B,),
        in_specs=[pl.BlockSpec((1, N, Din), lambda b: (b, 0, 0))] + adj_specs + [
            pl.BlockSpec(packed.shape, lambda b: (0, 0)),
        ],
        out_specs=pl.BlockSpec((1, N, D1), lambda b: (b, 0, 0)),
        compiler_params=pltpu.CompilerParams(
            dimension_semantics=("arbitrary",),
            vmem_limit_bytes=int((64 << 20) * 0.75)),
    )(node_features, *([adj] * R), packed)
